# baseline (speedup 1.0000x reference)
import functools

import jax
import jax.numpy as jnp
from jax import lax
from jax.experimental import pallas as pl
from jax.experimental.pallas import tpu as pltpu
from jax.experimental.pallas import tpu_sc as plsc

N_NODES = 10000
N_EDGES = 320000
IN_DIMS = 128
OUT_DIMS = 64
EPS = 1e-5

D = OUT_DIMS
NTILES = 16
EPT = N_EDGES // NTILES
NSEG = 10
SEG_E = EPT // NSEG
GRP_E = 1000
GPS = SEG_E // GRP_E
VPS = SEG_E // 16
ROWS_PT = 632
ROWS = ROWS_PT * NTILES


def _pre_body(x_ref, wl_ref, yp_ref, yn_ref):
    xw = jnp.dot(x_ref[...], wl_ref[...], preferred_element_type=jnp.float32)
    yp_ref[...] = xw[:, :OUT_DIMS]
    yn_ref[...] = xw[:, OUT_DIMS:]


def _pre(x, wl_cat):
    return pl.pallas_call(
        _pre_body,
        out_shape=[
            jax.ShapeDtypeStruct((N_NODES, D), jnp.float32),
            jax.ShapeDtypeStruct((N_NODES, D), jnp.float32),
        ],
        grid=(5,),
        in_specs=[
            pl.BlockSpec((N_NODES // 5, IN_DIMS), lambda i: (i, 0)),
            pl.BlockSpec((IN_DIMS, 2 * OUT_DIMS), lambda i: (0, 0)),
        ],
        out_specs=[
            pl.BlockSpec((N_NODES // 5, D), lambda i: (i, 0)),
            pl.BlockSpec((N_NODES // 5, D), lambda i: (i, 0)),
        ],
    )(x, wl_cat)


def _sc_body(yp, yn, pe, ne, zeros_hbm, outp, outn, histp, histn,
             src_v, dst_v, buf, hist, acc, sem):
    c = lax.axis_index("c")
    s = lax.axis_index("s")

    pltpu.sync_copy(zeros_hbm.at[pl.ds(s * ROWS_PT, ROWS_PT)],
                    acc.at[pl.ds(s * ROWS_PT, ROWS_PT)])

    zero16 = jnp.zeros((16,), jnp.float32)

    def zbody(i, carry):
        hist[pl.ds(i * 16, 16)] = zero16
        return carry

    lax.fori_loop(0, ROWS // 16, zbody, 0)
    ones16 = jnp.ones((16,), jnp.float32)
    plsc.subcore_barrier()

    def run(y_h, e_h, out_h, hist_h):
        def seg_body(k, carry):
            off = s * EPT + k * SEG_E
            pltpu.sync_copy(e_h.at[0, pl.ds(off, SEG_E)], src_v)
            pltpu.sync_copy(e_h.at[1, pl.ds(off, SEG_E)], dst_v)

            gat = pltpu.async_copy(
                y_h.at[src_v.at[pl.ds(0, GRP_E)]], buf, sem)

            def hbody(v, c3):
                idx16 = dst_v[pl.ds(v * 16, 16)]
                plsc.addupdate_scatter(hist, [idx16], ones16)
                return c3

            lax.fori_loop(0, VPS, hbody, 0)

            gat.wait()
            pltpu.sync_copy(
                buf, acc.at[dst_v.at[pl.ds(0, GRP_E)]], add=True)
            for g in range(1, GPS):
                pltpu.async_copy(
                    y_h.at[src_v.at[pl.ds(g * GRP_E, GRP_E)]], buf, sem
                ).wait()
                pltpu.sync_copy(
                    buf, acc.at[dst_v.at[pl.ds(g * GRP_E, GRP_E)]], add=True)
            return carry

        lax.fori_loop(0, NSEG, seg_body, 0)
        pltpu.sync_copy(hist, hist_h.at[s])
        plsc.subcore_barrier()
        pltpu.sync_copy(acc.at[pl.ds(s * ROWS_PT, ROWS_PT)],
                        out_h.at[pl.ds(s * ROWS_PT, ROWS_PT)])

    @pl.when(c == 0)
    def _():
        run(yp, pe, outp, histp)

    @pl.when(c == 1)
    def _():
        run(yn, ne, outn, histn)


_sc_agg = functools.partial(
    pl.kernel,
    _sc_body,
    out_type=[
        jax.ShapeDtypeStruct((ROWS, D), jnp.float32),
        jax.ShapeDtypeStruct((ROWS, D), jnp.float32),
        jax.ShapeDtypeStruct((NTILES, ROWS), jnp.float32),
        jax.ShapeDtypeStruct((NTILES, ROWS), jnp.float32),
    ],
    mesh=plsc.VectorSubcoreMesh(core_axis_name="c", subcore_axis_name="s"),
    compiler_params=pltpu.CompilerParams(use_tc_tiling_on_sc=False,
                                         needs_layout_passes=False),
    scratch_types=[
        pltpu.VMEM((SEG_E,), jnp.int32),
        pltpu.VMEM((SEG_E,), jnp.int32),
        pltpu.VMEM((GRP_E, D), jnp.float32),
        pltpu.VMEM((ROWS,), jnp.float32),
        pltpu.VMEM_SHARED((ROWS, D), jnp.float32),
        pltpu.SemaphoreType.DMA,
    ],
)()


def _post_body(x_ref, sp_ref, sn_ref, hp_ref, hn_ref, wr_ref, b_ref, g_ref,
               be_ref, out_ref):
    xr = jnp.dot(x_ref[...], wr_ref[...], preferred_element_type=jnp.float32)
    ones_c = jnp.ones((NTILES, 1), jnp.float32)
    dn = (((0,), (0,)), ((), ()))
    cntp = lax.dot_general(hp_ref[...], ones_c, dn,
                           preferred_element_type=jnp.float32)
    cntn = lax.dot_general(hn_ref[...], ones_c, dn,
                           preferred_element_type=jnp.float32)
    aggp = sp_ref[...] / jnp.maximum(cntp[:N_NODES], 1.0)
    aggn = sn_ref[...] / jnp.maximum(cntn[:N_NODES], 1.0)
    pre = jnp.concatenate([aggp, aggn], axis=1) + xr + b_ref[...]
    mu = jnp.mean(pre, axis=0, keepdims=True)
    var = jnp.mean(jnp.square(pre - mu), axis=0, keepdims=True)
    out = (pre - mu) * lax.rsqrt(var + EPS) * g_ref[...] + be_ref[...]
    out_ref[...] = jnp.maximum(out, 0.0)


def _post(x, sp, sn, hp, hn, wr_cat, b_cat, g_cat, be_cat):
    return pl.pallas_call(
        _post_body,
        out_shape=jax.ShapeDtypeStruct((N_NODES, 2 * OUT_DIMS), jnp.float32),
    )(x, sp, sn, hp, hn, wr_cat, b_cat, g_cat, be_cat)


def kernel(x, pos_edge_index, neg_edge_index, W_pos_l, W_pos_r, b_pos,
           W_neg_l, W_neg_r, b_neg, gamma, beta):
    pe = pos_edge_index.astype(jnp.int32)
    ne = neg_edge_index.astype(jnp.int32)
    wl_cat = jnp.concatenate([W_pos_l, W_neg_l], axis=1)
    wr_cat = jnp.concatenate([W_pos_r, W_neg_r], axis=1)
    b_cat = jnp.concatenate([b_pos, b_neg]).reshape(1, 2 * OUT_DIMS)
    g_cat = gamma.reshape(1, 2 * OUT_DIMS)
    be_cat = beta.reshape(1, 2 * OUT_DIMS)
    zeros_hbm = jnp.zeros((ROWS, D), jnp.float32)

    yp, yn = _pre(x, wl_cat)
    sp_full, sn_full, hp, hn = _sc_agg(yp, yn, pe, ne, zeros_hbm)
    sp = sp_full[:N_NODES]
    sn = sn_full[:N_NODES]
    return _post(x, sp, sn, hp, hn, wr_cat, b_cat, g_cat, be_cat)


# speedup vs baseline: 1.0210x; 1.0210x over previous
import functools

import jax
import jax.numpy as jnp
from jax import lax
from jax.experimental import pallas as pl
from jax.experimental.pallas import tpu as pltpu
from jax.experimental.pallas import tpu_sc as plsc

N_NODES = 10000
N_EDGES = 320000
IN_DIMS = 128
OUT_DIMS = 64
EPS = 1e-5

D = OUT_DIMS
NTILES = 16
EPT = N_EDGES // NTILES
NSEG = 5
SEG_E = EPT // NSEG
GRP_E = 800
GPS = SEG_E // GRP_E
VPG = GRP_E // 16
ROWS_PT = 632
ROWS = ROWS_PT * NTILES


def _pre_body(x_ref, wl_ref, yp_ref, yn_ref):
    xw = jnp.dot(x_ref[...], wl_ref[...], preferred_element_type=jnp.float32)
    yp_ref[...] = xw[:, :OUT_DIMS]
    yn_ref[...] = xw[:, OUT_DIMS:]


def _pre(x, wl_cat):
    return pl.pallas_call(
        _pre_body,
        out_shape=[
            jax.ShapeDtypeStruct((N_NODES, D), jnp.float32),
            jax.ShapeDtypeStruct((N_NODES, D), jnp.float32),
        ],
        grid=(5,),
        in_specs=[
            pl.BlockSpec((N_NODES // 5, IN_DIMS), lambda i: (i, 0)),
            pl.BlockSpec((IN_DIMS, 2 * OUT_DIMS), lambda i: (0, 0)),
        ],
        out_specs=[
            pl.BlockSpec((N_NODES // 5, D), lambda i: (i, 0)),
            pl.BlockSpec((N_NODES // 5, D), lambda i: (i, 0)),
        ],
    )(x, wl_cat)


def _sc_body(yp, yn, pe, ne, zeros_hbm, outp, outn, histp, histn,
             src_v, dst_v, buf, hist, acc, sem):
    c = lax.axis_index("c")
    s = lax.axis_index("s")

    pltpu.sync_copy(zeros_hbm.at[pl.ds(s * ROWS_PT, ROWS_PT)],
                    acc.at[pl.ds(s * ROWS_PT, ROWS_PT)])

    zero16 = jnp.zeros((16,), jnp.float32)

    def zbody(i, carry):
        hist[pl.ds(i * 16, 16)] = zero16
        return carry

    lax.fori_loop(0, ROWS // 16, zbody, 0)
    ones16 = jnp.ones((16,), jnp.float32)
    plsc.subcore_barrier()

    def run(y_h, e_h, out_h, hist_h):
        def seg_body(k, carry):
            off = s * EPT + k * SEG_E
            pltpu.sync_copy(e_h.at[0, pl.ds(off, SEG_E)], src_v)
            pltpu.sync_copy(e_h.at[1, pl.ds(off, SEG_E)], dst_v)

            def body(g, c2):
                gat = pltpu.async_copy(
                    y_h.at[src_v.at[pl.ds(g * GRP_E, GRP_E)]], buf, sem)

                def hbody(v, c3):
                    idx16 = dst_v[pl.ds(g * GRP_E + v * 16, 16)]
                    plsc.addupdate_scatter(hist, [idx16], ones16)
                    return c3

                lax.fori_loop(0, VPG, hbody, 0)

                gat.wait()
                pltpu.sync_copy(
                    buf, acc.at[dst_v.at[pl.ds(g * GRP_E, GRP_E)]], add=True)
                return c2

            lax.fori_loop(0, GPS, body, 0)
            return carry

        lax.fori_loop(0, NSEG, seg_body, 0)
        pltpu.sync_copy(hist, hist_h.at[s])
        plsc.subcore_barrier()
        pltpu.sync_copy(acc.at[pl.ds(s * ROWS_PT, ROWS_PT)],
                        out_h.at[pl.ds(s * ROWS_PT, ROWS_PT)])

    @pl.when(c == 0)
    def _():
        run(yp, pe, outp, histp)

    @pl.when(c == 1)
    def _():
        run(yn, ne, outn, histn)


_sc_agg = functools.partial(
    pl.kernel,
    _sc_body,
    out_type=[
        jax.ShapeDtypeStruct((ROWS, D), jnp.float32),
        jax.ShapeDtypeStruct((ROWS, D), jnp.float32),
        jax.ShapeDtypeStruct((NTILES, ROWS), jnp.float32),
        jax.ShapeDtypeStruct((NTILES, ROWS), jnp.float32),
    ],
    mesh=plsc.VectorSubcoreMesh(core_axis_name="c", subcore_axis_name="s"),
    compiler_params=pltpu.CompilerParams(use_tc_tiling_on_sc=False,
                                         needs_layout_passes=False),
    scratch_types=[
        pltpu.VMEM((SEG_E,), jnp.int32),
        pltpu.VMEM((SEG_E,), jnp.int32),
        pltpu.VMEM((GRP_E, D), jnp.float32),
        pltpu.VMEM((ROWS,), jnp.float32),
        pltpu.VMEM_SHARED((ROWS, D), jnp.float32),
        pltpu.SemaphoreType.DMA,
    ],
)()


def _post_body(x_ref, sp_ref, sn_ref, hp_ref, hn_ref, wr_ref, b_ref, g_ref,
               be_ref, out_ref):
    xr = jnp.dot(x_ref[...], wr_ref[...], preferred_element_type=jnp.float32)
    ones_c = jnp.ones((NTILES, 1), jnp.float32)
    dn = (((0,), (0,)), ((), ()))
    cntp = lax.dot_general(hp_ref[...], ones_c, dn,
                           preferred_element_type=jnp.float32)
    cntn = lax.dot_general(hn_ref[...], ones_c, dn,
                           preferred_element_type=jnp.float32)
    aggp = sp_ref[...] / jnp.maximum(cntp[:N_NODES], 1.0)
    aggn = sn_ref[...] / jnp.maximum(cntn[:N_NODES], 1.0)
    pre = jnp.concatenate([aggp, aggn], axis=1) + xr + b_ref[...]
    mu = jnp.mean(pre, axis=0, keepdims=True)
    var = jnp.mean(jnp.square(pre - mu), axis=0, keepdims=True)
    out = (pre - mu) * lax.rsqrt(var + EPS) * g_ref[...] + be_ref[...]
    out_ref[...] = jnp.maximum(out, 0.0)


def _post(x, sp, sn, hp, hn, wr_cat, b_cat, g_cat, be_cat):
    return pl.pallas_call(
        _post_body,
        out_shape=jax.ShapeDtypeStruct((N_NODES, 2 * OUT_DIMS), jnp.float32),
    )(x, sp, sn, hp, hn, wr_cat, b_cat, g_cat, be_cat)


def kernel(x, pos_edge_index, neg_edge_index, W_pos_l, W_pos_r, b_pos,
           W_neg_l, W_neg_r, b_neg, gamma, beta):
    pe = pos_edge_index.astype(jnp.int32)
    ne = neg_edge_index.astype(jnp.int32)
    wl_cat = jnp.concatenate([W_pos_l, W_neg_l], axis=1)
    wr_cat = jnp.concatenate([W_pos_r, W_neg_r], axis=1)
    b_cat = jnp.concatenate([b_pos, b_neg]).reshape(1, 2 * OUT_DIMS)
    g_cat = gamma.reshape(1, 2 * OUT_DIMS)
    be_cat = beta.reshape(1, 2 * OUT_DIMS)
    zeros_hbm = jnp.zeros((ROWS, D), jnp.float32)

    yp, yn = _pre(x, wl_cat)
    sp_full, sn_full, hp, hn = _sc_agg(yp, yn, pe, ne, zeros_hbm)
    sp = sp_full[:N_NODES]
    sn = sn_full[:N_NODES]
    return _post(x, sp, sn, hp, hn, wr_cat, b_cat, g_cat, be_cat)
